# baseline (speedup 1.0000x reference)
import jax
import jax.numpy as jnp
from jax import lax
from jax.experimental import pallas as pl
from jax.experimental.pallas import tpu as pltpu
from jax.experimental.pallas import tpu_sc as plsc

BATCH = 4096
SEQ = 200
EMBEDDING_DIM = 64
PAD_DIM = 128
NUM_INDICES = BATCH * SEQ
NUM_CORES = 2
NUM_SUBCORES = 16
NUM_WORKERS = NUM_CORES * NUM_SUBCORES
PER_WORKER = NUM_INDICES // NUM_WORKERS
CHUNK = 128


def kernel(token_ids, embedding):
    flat_ids = token_ids.reshape(NUM_INDICES)
    table128 = jnp.pad(embedding, ((0, 0), (0, PAD_DIM - EMBEDDING_DIM)))

    mesh = plsc.VectorSubcoreMesh(core_axis_name="c", subcore_axis_name="s")

    @pl.kernel(
        out_type=jax.ShapeDtypeStruct((NUM_INDICES, PAD_DIM), embedding.dtype),
        mesh=mesh,
        scratch_types=[
            pltpu.VMEM((CHUNK,), jnp.int32),
            pltpu.VMEM((CHUNK, PAD_DIM), jnp.float32),
            pltpu.SemaphoreType.DMA,
        ],
    )
    def gather_kernel(table_hbm, idx_hbm, out_hbm, idx_v, rows_v, sem):
        wid = lax.axis_index("s") * NUM_CORES + lax.axis_index("c")
        base = wid * PER_WORKER

        @pl.loop(0, PER_WORKER, step=CHUNK)
        def _(off):
            pltpu.sync_copy(idx_hbm.at[pl.ds(base + off, CHUNK)], idx_v)
            pltpu.async_copy(table_hbm.at[idx_v], rows_v, sem).wait()
            pltpu.sync_copy(rows_v, out_hbm.at[pl.ds(base + off, CHUNK)])

    out = gather_kernel(table128, flat_ids)
    return out[:, :EMBEDDING_DIM].reshape(BATCH, SEQ, EMBEDDING_DIM)


# speedup vs baseline: 1.2469x; 1.2469x over previous
import jax
import jax.numpy as jnp
from jax import lax
from jax.experimental import pallas as pl
from jax.experimental.pallas import tpu as pltpu
from jax.experimental.pallas import tpu_sc as plsc

BATCH = 4096
SEQ = 200
EMBEDDING_DIM = 64
PAD_DIM = 128
LANES = 16
NUM_INDICES = BATCH * SEQ
NUM_CORES = 2
NUM_SUBCORES = 16
NUM_WORKERS = NUM_CORES * NUM_SUBCORES
PER_WORKER = NUM_INDICES // NUM_WORKERS
CHUNK = 128
NBUF = 4
HBUF = 2
ROUNDS = PER_WORKER // (CHUNK * NBUF)


def kernel(token_ids, embedding):
    flat_ids = token_ids.reshape(NUM_INDICES)
    table128 = jnp.pad(embedding, ((0, 0), (0, PAD_DIM - EMBEDDING_DIM)))

    mesh = plsc.VectorSubcoreMesh(core_axis_name="c", subcore_axis_name="s")

    @pl.kernel(
        out_type=jax.ShapeDtypeStruct((NUM_INDICES, EMBEDDING_DIM), embedding.dtype),
        mesh=mesh,
        scratch_types=[
            pltpu.VMEM((NBUF, CHUNK), jnp.int32),
            pltpu.VMEM((NBUF, CHUNK, PAD_DIM), jnp.float32),
            pltpu.VMEM((HBUF, CHUNK, EMBEDDING_DIM), jnp.float32),
            pltpu.SemaphoreType.DMA((NBUF,)),
            pltpu.SemaphoreType.DMA((NBUF,)),
            pltpu.SemaphoreType.DMA((HBUF,)),
        ],
    )
    def gather_kernel(
        table_hbm, idx_hbm, out_hbm, idx_v, rows_v, half_v, sem_i, sem_g, sem_o
    ):
        wid = lax.axis_index("s") * NUM_CORES + lax.axis_index("c")
        base = wid * PER_WORKER

        for b in range(NBUF):
            pltpu.async_copy(
                idx_hbm.at[pl.ds(base + b * CHUNK, CHUNK)], idx_v.at[b], sem_i.at[b]
            )

        @pl.loop(0, ROUNDS)
        def _(r):
            g0 = base + r * (NBUF * CHUNK)

            for b in range(NBUF):
                pltpu.make_async_copy(
                    idx_hbm.at[pl.ds(g0 + b * CHUNK, CHUNK)],
                    idx_v.at[b],
                    sem_i.at[b],
                ).wait()
                pltpu.async_copy(
                    table_hbm.at[idx_v.at[b]], rows_v.at[b], sem_g.at[b]
                )

            for b in range(NBUF):
                start = g0 + b * CHUNK
                h = b % HBUF
                pltpu.make_async_copy(
                    table_hbm.at[idx_v.at[b]], rows_v.at[b], sem_g.at[b]
                ).wait()

                def drain():
                    pltpu.make_async_copy(
                        half_v.at[h],
                        out_hbm.at[pl.ds(base, CHUNK)],
                        sem_o.at[h],
                    ).wait()

                if b >= HBUF:
                    drain()
                else:
                    pl.when(r > 0)(drain)

                @pl.loop(0, CHUNK)
                def _(j):
                    for c in range(0, EMBEDDING_DIM, LANES):
                        half_v[h, j, pl.ds(c, LANES)] = rows_v[b, j, pl.ds(c, LANES)]

                pltpu.async_copy(
                    half_v.at[h], out_hbm.at[pl.ds(start, CHUNK)], sem_o.at[h]
                )

                @pl.when(r + 1 < ROUNDS)
                def _():
                    pltpu.async_copy(
                        idx_hbm.at[pl.ds(g0 + (NBUF + b) * CHUNK, CHUNK)],
                        idx_v.at[b],
                        sem_i.at[b],
                    )

        for h in range(HBUF):
            pltpu.make_async_copy(
                half_v.at[h], out_hbm.at[pl.ds(base, CHUNK)], sem_o.at[h]
            ).wait()

    out = gather_kernel(table128, flat_ids)
    return out.reshape(BATCH, SEQ, EMBEDDING_DIM)
